# baseline (speedup 1.0000x reference)
import functools

import jax
import jax.numpy as jnp
from jax import lax
from jax.experimental import pallas as pl
from jax.experimental.pallas import tpu as pltpu
from jax.experimental.pallas import tpu_sc as plsc

_NC, _NS = 2, 16
_NW = _NC * _NS
_GB = 10
_NJ = 3


def _mlp_msg_body(E, BE, ef_ref, xs_ref, W1_ref, b1_ref, W2_ref, b2_ref,
                  R_ref, S_ref, F_ref, out_ref):
    hid = jnp.maximum(
        jnp.dot(ef_ref[...], W1_ref[...], preferred_element_type=jnp.float32)
        + b1_ref[...], 0.0)
    we = jnp.dot(hid, W2_ref[...], preferred_element_type=jnp.float32) + b2_ref[...]
    xr = jnp.dot(xs_ref[...], R_ref[...], preferred_element_type=jnp.float32)
    msg = jnp.dot(xr * we, S_ref[...], preferred_element_type=jnp.float32)
    row = pl.program_id(0) * BE + lax.broadcasted_iota(jnp.int32, (BE, 1), 0)
    valid = (row < E).astype(jnp.float32)
    out_ref[...] = msg + valid * F_ref[...]


def _scatter_body(n_chunks, zrows_pt, msgv4, dst3, zrows, out, idx_v, val_v,
                  acc_sh, sem):
    cid = lax.axis_index("c")
    sid = lax.axis_index("s")
    wid = sid * _NC + cid
    pltpu.sync_copy(zrows, acc_sh.at[pl.ds(sid * zrows_pt, zrows_pt)])
    plsc.subcore_barrier()
    pltpu.sync_copy(dst3.at[wid], idx_v)

    def body(g, carry):
        pltpu.sync_copy(msgv4.at[wid, pl.ds(g * _GB, _GB)], val_v)
        handles = [
            pltpu.async_copy(val_v.at[b], acc_sh.at[idx_v.at[g * _GB + b]],
                             sem, add=True)
            for b in range(_GB)
        ]
        for hnd in handles:
            hnd.wait()
        return carry

    lax.fori_loop(0, n_chunks // _GB, body, 0)
    plsc.subcore_barrier()
    pltpu.sync_copy(acc_sh.at[pl.ds(sid * zrows_pt, zrows_pt)],
                    out.at[pl.ds(sid * zrows_pt, zrows_pt),
                           pl.ds(cid * 32, 32)])


def _idx_gather_body(nj, src_h, dst_h, eidx3, ef_h, out_si, out_di, out_e,
                     idx_v, sidx_v, didx_v, efx_v, sem):
    cid = lax.axis_index("c")
    sid = lax.axis_index("s")
    wid = sid * _NC + cid
    pltpu.sync_copy(eidx3.at[wid], idx_v)
    handles = []
    for j in range(nj):
        handles.append(pltpu.async_copy(src_h.at[idx_v.at[j]], sidx_v.at[j], sem))
        handles.append(pltpu.async_copy(dst_h.at[idx_v.at[j]], didx_v.at[j], sem))
        handles.append(pltpu.async_copy(ef_h.at[idx_v.at[j]], efx_v.at[j], sem))
    for hnd in handles:
        hnd.wait()
    pltpu.sync_copy(sidx_v, out_si.at[wid])
    pltpu.sync_copy(didx_v, out_di.at[wid])
    pltpu.sync_copy(efx_v, out_e.at[wid])


def _row_gather_body(nj, si3, di3, sp_h, out_s, out_d, sidx_v, didx_v, sps_v,
                     spd_v, sem):
    cid = lax.axis_index("c")
    sid = lax.axis_index("s")
    wid = sid * _NC + cid
    pltpu.sync_copy(si3.at[wid], sidx_v)
    pltpu.sync_copy(di3.at[wid], didx_v)
    handles = []
    for j in range(nj):
        handles.append(pltpu.async_copy(sp_h.at[sidx_v.at[j]], sps_v.at[j], sem))
        handles.append(pltpu.async_copy(sp_h.at[didx_v.at[j]], spd_v.at[j], sem))
    for hnd in handles:
        hnd.wait()
    pltpu.sync_copy(sps_v, out_s.at[wid])
    pltpu.sync_copy(spd_v, out_d.at[wid])


def _cls_body(sps_ref, spd_ref, ef_ref, A_ref, B_ref, C_ref, bias_ref,
              bc1_ref, Wc2_ref, bc2_ref, out_ref):
    def fin(sp):
        a = sp[:, :32] + sp[:, 32:]
        return jnp.maximum(
            a[:, :16] / jnp.maximum(a[:, 16:17], 1.0) + bias_ref[...], 0.0)

    sh = fin(sps_ref[...])
    dh = fin(spd_ref[...])
    z = jnp.maximum(
        jnp.dot(sh, A_ref[...], preferred_element_type=jnp.float32)
        + jnp.dot(dh, B_ref[...], preferred_element_type=jnp.float32)
        + jnp.dot(ef_ref[...], C_ref[...], preferred_element_type=jnp.float32)
        + bc1_ref[...], 0.0)
    out_ref[...] = jnp.dot(z, Wc2_ref[...],
                           preferred_element_type=jnp.float32) + bc2_ref[...]


def kernel(node_feats, edge_feats, edge_index, edge_indices, W1, b1, W2, b2,
           conv_bias, Wc1, bc1, Wc2, bc2):
    N, IN = node_feats.shape
    E, EF = edge_feats.shape
    H = conv_bias.shape[0]
    K = W1.shape[1]
    HI = W2.shape[1]
    NSUP = edge_indices.shape[0]
    OUT = Wc2.shape[1]
    HW = 32

    src = edge_index[0]
    dst = edge_index[1]

    NS_pad = _NW * _NJ * 128
    eidx3 = jnp.pad(edge_indices, (0, NS_pad - NSUP)).reshape(_NW, _NJ, 128)
    idxgather = functools.partial(
        pl.kernel,
        mesh=plsc.VectorSubcoreMesh(core_axis_name="c", subcore_axis_name="s"),
        compiler_params=pltpu.CompilerParams(use_tc_tiling_on_sc=False),
        out_type=[
            jax.ShapeDtypeStruct((_NW, _NJ, 128), jnp.int32),
            jax.ShapeDtypeStruct((_NW, _NJ, 128), jnp.int32),
            jax.ShapeDtypeStruct((_NW, _NJ, 128, EF), jnp.float32),
        ],
        scratch_types=[
            pltpu.VMEM((_NJ, 128), jnp.int32),
            pltpu.VMEM((_NJ, 128), jnp.int32),
            pltpu.VMEM((_NJ, 128), jnp.int32),
            pltpu.VMEM((_NJ, 128, EF), jnp.float32),
            pltpu.SemaphoreType.DMA,
        ],
    )(functools.partial(_idx_gather_body, _NJ))
    si3, di3, out_e = idxgather(src, dst, eidx3, edge_feats)

    BE = 1024
    E_pad = ((E + _NW * 128 - 1) // (_NW * 128)) * (_NW * 128)
    assert E_pad % BE == 0
    rows_per_tile = E_pad // _NW
    n_chunks = rows_per_tile // 128

    R = (jnp.arange(HI)[None, :] // H == jnp.arange(IN)[:, None]).astype(jnp.float32)
    S = (jnp.arange(HI)[:, None] % H == jnp.arange(H)[None, :]).astype(jnp.float32)
    S = jnp.concatenate([S, jnp.zeros((HI, HW - H), jnp.float32)], axis=1)
    F = (jnp.arange(HW)[None, :] == H).astype(jnp.float32)

    xs = node_feats[src]
    xs_p = jnp.pad(xs, ((0, E_pad - E), (0, 0)))
    ef_p = jnp.pad(edge_feats, ((0, E_pad - E), (0, 0)))

    grid = (E_pad // BE,)
    msgv = pl.pallas_call(
        functools.partial(_mlp_msg_body, E, BE),
        grid=grid,
        in_specs=[
            pl.BlockSpec((BE, EF), lambda i: (i, 0)),
            pl.BlockSpec((BE, IN), lambda i: (i, 0)),
            pl.BlockSpec((EF, K), lambda i: (0, 0)),
            pl.BlockSpec((1, K), lambda i: (0, 0)),
            pl.BlockSpec((K, HI), lambda i: (0, 0)),
            pl.BlockSpec((1, HI), lambda i: (0, 0)),
            pl.BlockSpec((IN, HI), lambda i: (0, 0)),
            pl.BlockSpec((HI, HW), lambda i: (0, 0)),
            pl.BlockSpec((1, HW), lambda i: (0, 0)),
        ],
        out_specs=pl.BlockSpec((BE, HW), lambda i: (i, 0)),
        out_shape=jax.ShapeDtypeStruct((E_pad, HW), jnp.float32),
    )(ef_p, xs_p, W1, b1.reshape(1, K), W2, b2.reshape(1, HI), R, S, F)

    N_pad = ((N + _NS * 8 - 1) // (_NS * 8)) * (_NS * 8)
    zrows_pt = N_pad // _NS
    dst3 = jnp.pad(dst, (0, E_pad - E)).reshape(_NW, n_chunks, 128)
    msgv4 = msgv.reshape(_NW, n_chunks, 128, HW)
    zrows = jnp.zeros((zrows_pt, HW), jnp.float32)

    scatter = functools.partial(
        pl.kernel,
        mesh=plsc.VectorSubcoreMesh(core_axis_name="c", subcore_axis_name="s"),
        compiler_params=pltpu.CompilerParams(use_tc_tiling_on_sc=False),
        out_type=jax.ShapeDtypeStruct((N_pad, 2 * HW), jnp.float32),
        scratch_types=[
            pltpu.VMEM((n_chunks, 128), jnp.int32),
            pltpu.VMEM((_GB, 128, HW), jnp.float32),
            pltpu.VMEM_SHARED((N_pad, HW), jnp.float32),
            pltpu.SemaphoreType.DMA,
        ],
    )(functools.partial(_scatter_body, n_chunks, zrows_pt))
    sp = scatter(msgv4, dst3, zrows)

    rowgather = functools.partial(
        pl.kernel,
        mesh=plsc.VectorSubcoreMesh(core_axis_name="c", subcore_axis_name="s"),
        compiler_params=pltpu.CompilerParams(use_tc_tiling_on_sc=False),
        out_type=[
            jax.ShapeDtypeStruct((_NW, _NJ, 128, 2 * HW), jnp.float32),
            jax.ShapeDtypeStruct((_NW, _NJ, 128, 2 * HW), jnp.float32),
        ],
        scratch_types=[
            pltpu.VMEM((_NJ, 128), jnp.int32),
            pltpu.VMEM((_NJ, 128), jnp.int32),
            pltpu.VMEM((_NJ, 128, 2 * HW), jnp.float32),
            pltpu.VMEM((_NJ, 128, 2 * HW), jnp.float32),
            pltpu.SemaphoreType.DMA,
        ],
    )(functools.partial(_row_gather_body, _NJ))
    out_s, out_d = rowgather(si3, di3, sp)
    sps = out_s.reshape(NS_pad, 2 * HW)
    spd = out_d.reshape(NS_pad, 2 * HW)
    efx = out_e.reshape(NS_pad, EF)

    logits_p = pl.pallas_call(
        _cls_body,
        in_specs=[
            pl.BlockSpec((NS_pad, 2 * HW), lambda: (0, 0)),
            pl.BlockSpec((NS_pad, 2 * HW), lambda: (0, 0)),
            pl.BlockSpec((NS_pad, EF), lambda: (0, 0)),
            pl.BlockSpec((H, H), lambda: (0, 0)),
            pl.BlockSpec((H, H), lambda: (0, 0)),
            pl.BlockSpec((EF, H), lambda: (0, 0)),
            pl.BlockSpec((1, H), lambda: (0, 0)),
            pl.BlockSpec((1, H), lambda: (0, 0)),
            pl.BlockSpec((H, OUT), lambda: (0, 0)),
            pl.BlockSpec((1, OUT), lambda: (0, 0)),
        ],
        out_specs=pl.BlockSpec((NS_pad, OUT), lambda: (0, 0)),
        out_shape=jax.ShapeDtypeStruct((NS_pad, OUT), jnp.float32),
    )(sps, spd, efx, Wc1[:H], Wc1[H:2 * H], Wc1[2 * H:],
      conv_bias.reshape(1, H), bc1.reshape(1, H), Wc2, bc2.reshape(1, OUT))
    return logits_p[:NSUP]


# speedup vs baseline: 1.0718x; 1.0718x over previous
import functools

import jax
import jax.numpy as jnp
from jax import lax
from jax.experimental import pallas as pl
from jax.experimental.pallas import tpu as pltpu
from jax.experimental.pallas import tpu_sc as plsc

_NC, _NS = 2, 16
_NW = _NC * _NS
_GB = 5
_NJ = 6


def _mlp_msg_body(E, BE, ef_ref, xs_ref, W1_ref, b1_ref, W2_ref, b2_ref,
                  R_ref, S_ref, F_ref, out_ref):
    hid = jnp.maximum(
        jnp.dot(ef_ref[...], W1_ref[...], preferred_element_type=jnp.float32)
        + b1_ref[...], 0.0)
    we = jnp.dot(hid, W2_ref[...], preferred_element_type=jnp.float32) + b2_ref[...]
    xr = jnp.dot(xs_ref[...], R_ref[...], preferred_element_type=jnp.float32)
    msg = jnp.dot(xr * we, S_ref[...], preferred_element_type=jnp.float32)
    row = pl.program_id(0) * BE + lax.broadcasted_iota(jnp.int32, (BE, 1), 0)
    valid = (row < E).astype(jnp.float32)
    out_ref[...] = msg + valid * F_ref[...]


def _agg_body(n_groups, zrows_pt, nj, msgv4, dst3, zrows, eidx3, src_h, dst_h,
              ef_h, out_s, out_d, out_e, idx_v, val_v, acc_sh, eidx_v, sidx_v,
              didx_v, efx_v, sbuf_v, dbuf_v, sem_i, sem_s, sem_r):
    cid = lax.axis_index("c")
    sid = lax.axis_index("s")
    wid = sid * _NC + cid
    pltpu.sync_copy(eidx3.at[sid], eidx_v)
    ih = []
    for j in range(nj):
        ih.append(pltpu.async_copy(src_h.at[eidx_v.at[j]], sidx_v.at[j], sem_i))
        ih.append(pltpu.async_copy(dst_h.at[eidx_v.at[j]], didx_v.at[j], sem_i))
        ih.append(pltpu.async_copy(ef_h.at[eidx_v.at[j]], efx_v.at[j], sem_i))
    pltpu.sync_copy(zrows, acc_sh.at[pl.ds(sid * zrows_pt, zrows_pt)])
    pltpu.sync_copy(dst3.at[wid], idx_v)
    plsc.subcore_barrier()

    def body(g, carry):
        pltpu.sync_copy(msgv4.at[wid, pl.ds(g * _GB, _GB)], val_v)
        handles = [
            pltpu.async_copy(val_v.at[b], acc_sh.at[idx_v.at[g * _GB + b]],
                             sem_s, add=True)
            for b in range(_GB)
        ]
        for hnd in handles:
            hnd.wait()
        return carry

    lax.fori_loop(0, n_groups, body, 0)
    for hnd in ih:
        hnd.wait()
    plsc.subcore_barrier()

    for j in range(nj):
        h1 = pltpu.async_copy(acc_sh.at[sidx_v.at[j]], sbuf_v, sem_r)
        h2 = pltpu.async_copy(acc_sh.at[didx_v.at[j]], dbuf_v, sem_r)
        h1.wait()
        h2.wait()
        pltpu.sync_copy(sbuf_v, out_s.at[sid, j, pl.ds(0, 128),
                                         pl.ds(cid * 32, 32)])
        pltpu.sync_copy(dbuf_v, out_d.at[sid, j, pl.ds(0, 128),
                                         pl.ds(cid * 32, 32)])

    @pl.when(cid == 0)
    def _():
        pltpu.sync_copy(efx_v, out_e.at[sid])


def _cls_body(sps_ref, spd_ref, ef_ref, A_ref, B_ref, C_ref, bias_ref,
              bc1_ref, Wc2_ref, bc2_ref, out_ref):
    def fin(sp):
        a = sp[:, :32] + sp[:, 32:]
        return jnp.maximum(
            a[:, :16] / jnp.maximum(a[:, 16:17], 1.0) + bias_ref[...], 0.0)

    sh = fin(sps_ref[...])
    dh = fin(spd_ref[...])
    z = jnp.maximum(
        jnp.dot(sh, A_ref[...], preferred_element_type=jnp.float32)
        + jnp.dot(dh, B_ref[...], preferred_element_type=jnp.float32)
        + jnp.dot(ef_ref[...], C_ref[...], preferred_element_type=jnp.float32)
        + bc1_ref[...], 0.0)
    out_ref[...] = jnp.dot(z, Wc2_ref[...],
                           preferred_element_type=jnp.float32) + bc2_ref[...]


def kernel(node_feats, edge_feats, edge_index, edge_indices, W1, b1, W2, b2,
           conv_bias, Wc1, bc1, Wc2, bc2):
    N, IN = node_feats.shape
    E, EF = edge_feats.shape
    H = conv_bias.shape[0]
    K = W1.shape[1]
    HI = W2.shape[1]
    NSUP = edge_indices.shape[0]
    OUT = Wc2.shape[1]
    HW = 32

    src = edge_index[0]
    dst = edge_index[1]

    BE = 1024
    E_pad = ((E + _NW * 128 - 1) // (_NW * 128)) * (_NW * 128)
    assert E_pad % BE == 0
    rows_per_tile = E_pad // _NW
    n_chunks = rows_per_tile // 128

    R = (jnp.arange(HI)[None, :] // H == jnp.arange(IN)[:, None]).astype(jnp.float32)
    S = (jnp.arange(HI)[:, None] % H == jnp.arange(H)[None, :]).astype(jnp.float32)
    S = jnp.concatenate([S, jnp.zeros((HI, HW - H), jnp.float32)], axis=1)
    F = (jnp.arange(HW)[None, :] == H).astype(jnp.float32)

    xs = node_feats[src]
    xs_p = jnp.pad(xs, ((0, E_pad - E), (0, 0)))
    ef_p = jnp.pad(edge_feats, ((0, E_pad - E), (0, 0)))

    grid = (E_pad // BE,)
    msgv = pl.pallas_call(
        functools.partial(_mlp_msg_body, E, BE),
        grid=grid,
        in_specs=[
            pl.BlockSpec((BE, EF), lambda i: (i, 0)),
            pl.BlockSpec((BE, IN), lambda i: (i, 0)),
            pl.BlockSpec((EF, K), lambda i: (0, 0)),
            pl.BlockSpec((1, K), lambda i: (0, 0)),
            pl.BlockSpec((K, HI), lambda i: (0, 0)),
            pl.BlockSpec((1, HI), lambda i: (0, 0)),
            pl.BlockSpec((IN, HI), lambda i: (0, 0)),
            pl.BlockSpec((HI, HW), lambda i: (0, 0)),
            pl.BlockSpec((1, HW), lambda i: (0, 0)),
        ],
        out_specs=pl.BlockSpec((BE, HW), lambda i: (i, 0)),
        out_shape=jax.ShapeDtypeStruct((E_pad, HW), jnp.float32),
    )(ef_p, xs_p, W1, b1.reshape(1, K), W2, b2.reshape(1, HI), R, S, F)

    N_pad = ((N + _NS * 8 - 1) // (_NS * 8)) * (_NS * 8)
    zrows_pt = N_pad // _NS
    dst3 = jnp.pad(dst, (0, E_pad - E)).reshape(_NW, n_chunks, 128)
    msgv4 = msgv.reshape(_NW, n_chunks, 128, HW)
    zrows = jnp.zeros((zrows_pt, HW), jnp.float32)

    NS_pad = _NS * _NJ * 128
    eidx3 = jnp.pad(edge_indices, (0, NS_pad - NSUP)).reshape(_NS, _NJ, 128)
    agg = functools.partial(
        pl.kernel,
        mesh=plsc.VectorSubcoreMesh(core_axis_name="c", subcore_axis_name="s"),
        compiler_params=pltpu.CompilerParams(use_tc_tiling_on_sc=False),
        out_type=[
            jax.ShapeDtypeStruct((_NS, _NJ, 128, 2 * HW), jnp.float32),
            jax.ShapeDtypeStruct((_NS, _NJ, 128, 2 * HW), jnp.float32),
            jax.ShapeDtypeStruct((_NS, _NJ, 128, EF), jnp.float32),
        ],
        scratch_types=[
            pltpu.VMEM((n_chunks, 128), jnp.int32),
            pltpu.VMEM((_GB, 128, HW), jnp.float32),
            pltpu.VMEM_SHARED((N_pad, HW), jnp.float32),
            pltpu.VMEM((_NJ, 128), jnp.int32),
            pltpu.VMEM((_NJ, 128), jnp.int32),
            pltpu.VMEM((_NJ, 128), jnp.int32),
            pltpu.VMEM((_NJ, 128, EF), jnp.float32),
            pltpu.VMEM((128, HW), jnp.float32),
            pltpu.VMEM((128, HW), jnp.float32),
            pltpu.SemaphoreType.DMA,
            pltpu.SemaphoreType.DMA,
            pltpu.SemaphoreType.DMA,
        ],
    )(functools.partial(_agg_body, n_chunks // _GB, zrows_pt, _NJ))
    out_s, out_d, out_e = agg(msgv4, dst3, zrows, eidx3, src, dst, edge_feats)
    sps = out_s.reshape(NS_pad, 2 * HW)
    spd = out_d.reshape(NS_pad, 2 * HW)
    efx = out_e.reshape(NS_pad, EF)

    logits_p = pl.pallas_call(
        _cls_body,
        in_specs=[
            pl.BlockSpec((NS_pad, 2 * HW), lambda: (0, 0)),
            pl.BlockSpec((NS_pad, 2 * HW), lambda: (0, 0)),
            pl.BlockSpec((NS_pad, EF), lambda: (0, 0)),
            pl.BlockSpec((H, H), lambda: (0, 0)),
            pl.BlockSpec((H, H), lambda: (0, 0)),
            pl.BlockSpec((EF, H), lambda: (0, 0)),
            pl.BlockSpec((1, H), lambda: (0, 0)),
            pl.BlockSpec((1, H), lambda: (0, 0)),
            pl.BlockSpec((H, OUT), lambda: (0, 0)),
            pl.BlockSpec((1, OUT), lambda: (0, 0)),
        ],
        out_specs=pl.BlockSpec((NS_pad, OUT), lambda: (0, 0)),
        out_shape=jax.ShapeDtypeStruct((NS_pad, OUT), jnp.float32),
    )(sps, spd, efx, Wc1[:H], Wc1[H:2 * H], Wc1[2 * H:],
      conv_bias.reshape(1, H), bc1.reshape(1, H), Wc2, bc2.reshape(1, OUT))
    return logits_p[:NSUP]


# speedup vs baseline: 1.4299x; 1.3340x over previous
import functools

import jax
import jax.numpy as jnp
from jax import lax
from jax.experimental import pallas as pl
from jax.experimental.pallas import tpu as pltpu
from jax.experimental.pallas import tpu_sc as plsc

_NC, _NS = 2, 16
_NW = _NC * _NS
_GB = 5
_GBX = 6
_NJ = 6


def _mlp_msg_body(E, BE, ef_ref, xs_ref, W1_ref, b1_ref, W2_ref, b2_ref,
                  R_ref, S_ref, F_ref, out_ref):
    hid = jnp.maximum(
        jnp.dot(ef_ref[...], W1_ref[...], preferred_element_type=jnp.float32)
        + b1_ref[...], 0.0)
    we = jnp.dot(hid, W2_ref[...], preferred_element_type=jnp.float32) + b2_ref[...]
    xr = jnp.dot(xs_ref[...], R_ref[...], preferred_element_type=jnp.float32)
    msg = jnp.dot(xr * we, S_ref[...], preferred_element_type=jnp.float32)
    row = pl.program_id(0) * BE + lax.broadcasted_iota(jnp.int32, (BE, 1), 0)
    valid = (row < E).astype(jnp.float32)
    out_ref[...] = (msg + F_ref[...]) * valid


def _xs_gather_body(n_groups, nf_h, src3, out, idx_v, buf_v, sem):
    cid = lax.axis_index("c")
    sid = lax.axis_index("s")
    wid = sid * _NC + cid
    pltpu.sync_copy(src3.at[wid], idx_v)

    def body(g, carry):
        handles = [
            pltpu.async_copy(nf_h.at[idx_v.at[g * _GBX + b]], buf_v.at[b], sem)
            for b in range(_GBX)
        ]
        for hnd in handles:
            hnd.wait()
        pltpu.sync_copy(buf_v, out.at[wid, pl.ds(g * _GBX, _GBX)])
        return carry

    lax.fori_loop(0, n_groups, body, 0)


def _agg_body(n_groups, zrows_pt, nj, msgv4, dst3, zrows, eidx3, src_h, dst_h,
              ef_h, out_s, out_d, out_e, idx_v, val_v, acc_sh, eidx_v, sidx_v,
              didx_v, efx_v, sbuf_v, dbuf_v, sem_i, sem_s, sem_r):
    cid = lax.axis_index("c")
    sid = lax.axis_index("s")
    wid = sid * _NC + cid
    pltpu.sync_copy(eidx3.at[sid], eidx_v)
    ih = []
    for j in range(nj):
        ih.append(pltpu.async_copy(src_h.at[eidx_v.at[j]], sidx_v.at[j], sem_i))
        ih.append(pltpu.async_copy(dst_h.at[eidx_v.at[j]], didx_v.at[j], sem_i))
        ih.append(pltpu.async_copy(ef_h.at[eidx_v.at[j]], efx_v.at[j], sem_i))
    pltpu.sync_copy(zrows, acc_sh.at[pl.ds(sid * zrows_pt, zrows_pt)])
    pltpu.sync_copy(dst3.at[wid], idx_v)
    plsc.subcore_barrier()

    def body(g, carry):
        pltpu.sync_copy(msgv4.at[wid, pl.ds(g * _GB, _GB)], val_v)
        handles = [
            pltpu.async_copy(val_v.at[b], acc_sh.at[idx_v.at[g * _GB + b]],
                             sem_s, add=True)
            for b in range(_GB)
        ]
        for hnd in handles:
            hnd.wait()
        return carry

    lax.fori_loop(0, n_groups, body, 0)
    for hnd in ih:
        hnd.wait()
    plsc.subcore_barrier()

    for j in range(nj):
        h1 = pltpu.async_copy(acc_sh.at[sidx_v.at[j]], sbuf_v, sem_r)
        h2 = pltpu.async_copy(acc_sh.at[didx_v.at[j]], dbuf_v, sem_r)
        h1.wait()
        h2.wait()
        pltpu.sync_copy(sbuf_v, out_s.at[sid, j, pl.ds(0, 128),
                                         pl.ds(cid * 32, 32)])
        pltpu.sync_copy(dbuf_v, out_d.at[sid, j, pl.ds(0, 128),
                                         pl.ds(cid * 32, 32)])

    @pl.when(cid == 0)
    def _():
        pltpu.sync_copy(efx_v, out_e.at[sid])


def _cls_body(sps_ref, spd_ref, ef_ref, A_ref, B_ref, C_ref, bias_ref,
              bc1_ref, Wc2_ref, bc2_ref, out_ref):
    def fin(sp):
        a = sp[:, :32] + sp[:, 32:]
        return jnp.maximum(
            a[:, :16] / jnp.maximum(a[:, 16:17], 1.0) + bias_ref[...], 0.0)

    sh = fin(sps_ref[...])
    dh = fin(spd_ref[...])
    z = jnp.maximum(
        jnp.dot(sh, A_ref[...], preferred_element_type=jnp.float32)
        + jnp.dot(dh, B_ref[...], preferred_element_type=jnp.float32)
        + jnp.dot(ef_ref[...], C_ref[...], preferred_element_type=jnp.float32)
        + bc1_ref[...], 0.0)
    out_ref[...] = jnp.dot(z, Wc2_ref[...],
                           preferred_element_type=jnp.float32) + bc2_ref[...]


def kernel(node_feats, edge_feats, edge_index, edge_indices, W1, b1, W2, b2,
           conv_bias, Wc1, bc1, Wc2, bc2):
    N, IN = node_feats.shape
    E, EF = edge_feats.shape
    H = conv_bias.shape[0]
    K = W1.shape[1]
    HI = W2.shape[1]
    NSUP = edge_indices.shape[0]
    OUT = Wc2.shape[1]
    HW = 32

    src = edge_index[0]
    dst = edge_index[1]

    BE = 1024
    E_pad = ((E + _NW * 128 - 1) // (_NW * 128)) * (_NW * 128)
    assert E_pad % BE == 0
    rows_per_tile = E_pad // _NW
    n_chunks = rows_per_tile // 128

    R = (jnp.arange(HI)[None, :] // H == jnp.arange(IN)[:, None]).astype(jnp.float32)
    S = (jnp.arange(HI)[:, None] % H == jnp.arange(H)[None, :]).astype(jnp.float32)
    S = jnp.concatenate([S, jnp.zeros((HI, HW - H), jnp.float32)], axis=1)
    F = (jnp.arange(HW)[None, :] == H).astype(jnp.float32)

    src3 = jnp.pad(src, (0, E_pad - E)).reshape(_NW, n_chunks, 128)
    xsgather = functools.partial(
        pl.kernel,
        mesh=plsc.VectorSubcoreMesh(core_axis_name="c", subcore_axis_name="s"),
        compiler_params=pltpu.CompilerParams(use_tc_tiling_on_sc=False),
        out_type=jax.ShapeDtypeStruct((_NW, n_chunks, 128, IN), jnp.float32),
        scratch_types=[
            pltpu.VMEM((n_chunks, 128), jnp.int32),
            pltpu.VMEM((_GBX, 128, IN), jnp.float32),
            pltpu.SemaphoreType.DMA,
        ],
    )(functools.partial(_xs_gather_body, n_chunks // _GBX))
    xs_p = xsgather(node_feats, src3).reshape(E_pad, IN)
    ef_p = jnp.pad(edge_feats, ((0, E_pad - E), (0, 0)))

    grid = (E_pad // BE,)
    msgv = pl.pallas_call(
        functools.partial(_mlp_msg_body, E, BE),
        grid=grid,
        in_specs=[
            pl.BlockSpec((BE, EF), lambda i: (i, 0)),
            pl.BlockSpec((BE, IN), lambda i: (i, 0)),
            pl.BlockSpec((EF, K), lambda i: (0, 0)),
            pl.BlockSpec((1, K), lambda i: (0, 0)),
            pl.BlockSpec((K, HI), lambda i: (0, 0)),
            pl.BlockSpec((1, HI), lambda i: (0, 0)),
            pl.BlockSpec((IN, HI), lambda i: (0, 0)),
            pl.BlockSpec((HI, HW), lambda i: (0, 0)),
            pl.BlockSpec((1, HW), lambda i: (0, 0)),
        ],
        out_specs=pl.BlockSpec((BE, HW), lambda i: (i, 0)),
        out_shape=jax.ShapeDtypeStruct((E_pad, HW), jnp.float32),
    )(ef_p, xs_p, W1, b1.reshape(1, K), W2, b2.reshape(1, HI), R, S, F)

    N_pad = ((N + _NS * 8 - 1) // (_NS * 8)) * (_NS * 8)
    zrows_pt = N_pad // _NS
    dst3 = jnp.pad(dst, (0, E_pad - E)).reshape(_NW, n_chunks, 128)
    msgv4 = msgv.reshape(_NW, n_chunks, 128, HW)
    zrows = jnp.zeros((zrows_pt, HW), jnp.float32)

    NS_pad = _NS * _NJ * 128
    eidx3 = jnp.pad(edge_indices, (0, NS_pad - NSUP)).reshape(_NS, _NJ, 128)
    agg = functools.partial(
        pl.kernel,
        mesh=plsc.VectorSubcoreMesh(core_axis_name="c", subcore_axis_name="s"),
        compiler_params=pltpu.CompilerParams(use_tc_tiling_on_sc=False),
        out_type=[
            jax.ShapeDtypeStruct((_NS, _NJ, 128, 2 * HW), jnp.float32),
            jax.ShapeDtypeStruct((_NS, _NJ, 128, 2 * HW), jnp.float32),
            jax.ShapeDtypeStruct((_NS, _NJ, 128, EF), jnp.float32),
        ],
        scratch_types=[
            pltpu.VMEM((n_chunks, 128), jnp.int32),
            pltpu.VMEM((_GB, 128, HW), jnp.float32),
            pltpu.VMEM_SHARED((N_pad, HW), jnp.float32),
            pltpu.VMEM((_NJ, 128), jnp.int32),
            pltpu.VMEM((_NJ, 128), jnp.int32),
            pltpu.VMEM((_NJ, 128), jnp.int32),
            pltpu.VMEM((_NJ, 128, EF), jnp.float32),
            pltpu.VMEM((128, HW), jnp.float32),
            pltpu.VMEM((128, HW), jnp.float32),
            pltpu.SemaphoreType.DMA,
            pltpu.SemaphoreType.DMA,
            pltpu.SemaphoreType.DMA,
        ],
    )(functools.partial(_agg_body, n_chunks // _GB, zrows_pt, _NJ))
    out_s, out_d, out_e = agg(msgv4, dst3, zrows, eidx3, src, dst, edge_feats)
    sps = out_s.reshape(NS_pad, 2 * HW)
    spd = out_d.reshape(NS_pad, 2 * HW)
    efx = out_e.reshape(NS_pad, EF)

    logits_p = pl.pallas_call(
        _cls_body,
        in_specs=[
            pl.BlockSpec((NS_pad, 2 * HW), lambda: (0, 0)),
            pl.BlockSpec((NS_pad, 2 * HW), lambda: (0, 0)),
            pl.BlockSpec((NS_pad, EF), lambda: (0, 0)),
            pl.BlockSpec((H, H), lambda: (0, 0)),
            pl.BlockSpec((H, H), lambda: (0, 0)),
            pl.BlockSpec((EF, H), lambda: (0, 0)),
            pl.BlockSpec((1, H), lambda: (0, 0)),
            pl.BlockSpec((1, H), lambda: (0, 0)),
            pl.BlockSpec((H, OUT), lambda: (0, 0)),
            pl.BlockSpec((1, OUT), lambda: (0, 0)),
        ],
        out_specs=pl.BlockSpec((NS_pad, OUT), lambda: (0, 0)),
        out_shape=jax.ShapeDtypeStruct((NS_pad, OUT), jnp.float32),
    )(sps, spd, efx, Wc1[:H], Wc1[H:2 * H], Wc1[2 * H:],
      conv_bias.reshape(1, H), bc1.reshape(1, H), Wc2, bc2.reshape(1, OUT))
    return logits_p[:NSUP]


# speedup vs baseline: 1.4671x; 1.0261x over previous
import functools

import jax
import jax.numpy as jnp
from jax import lax
from jax.experimental import pallas as pl
from jax.experimental.pallas import tpu as pltpu
from jax.experimental.pallas import tpu_sc as plsc

_NC, _NS = 2, 16
_NW = _NC * _NS
_GB = 5
_GBX = 6
_NJ = 6


def _mlp_msg_body(E, BE, ef_ref, xs_ref, W1_ref, b1_ref, W2_ref, b2_ref,
                  R_ref, S_ref, F_ref, out_ref):
    hid = jnp.maximum(
        jnp.dot(ef_ref[...], W1_ref[...], preferred_element_type=jnp.float32)
        + b1_ref[...], 0.0)
    we = jnp.dot(hid.astype(jnp.bfloat16), W2_ref[...],
                 preferred_element_type=jnp.float32) + b2_ref[...]
    xr = jnp.dot(xs_ref[...], R_ref[...], preferred_element_type=jnp.float32)
    msg = jnp.dot((xr * we).astype(jnp.bfloat16), S_ref[...],
                  preferred_element_type=jnp.float32)
    row = pl.program_id(0) * BE + lax.broadcasted_iota(jnp.int32, (BE, 1), 0)
    valid = (row < E).astype(jnp.float32)
    out_ref[...] = (msg + F_ref[...]) * valid


def _xs_gather_body(n_groups, nf_h, src3, out, idx_v, buf_v, sem):
    cid = lax.axis_index("c")
    sid = lax.axis_index("s")
    wid = sid * _NC + cid
    pltpu.sync_copy(src3.at[wid], idx_v)

    def body(g, carry):
        handles = [
            pltpu.async_copy(nf_h.at[idx_v.at[g * _GBX + b]], buf_v.at[b], sem)
            for b in range(_GBX)
        ]
        for hnd in handles:
            hnd.wait()
        pltpu.sync_copy(buf_v, out.at[wid, pl.ds(g * _GBX, _GBX)])
        return carry

    lax.fori_loop(0, n_groups, body, 0)


def _agg_body(n_groups, zrows_pt, nj, msgv4, dst3, zrows, eidx3, src_h, dst_h,
              ef_h, out_s, out_d, out_e, idx_v, val_v, acc_sh, eidx_v, sidx_v,
              didx_v, efx_v, sbuf_v, dbuf_v, sem_i, sem_s, sem_r):
    cid = lax.axis_index("c")
    sid = lax.axis_index("s")
    wid = sid * _NC + cid
    pltpu.sync_copy(eidx3.at[sid], eidx_v)
    ih = []
    for j in range(nj):
        ih.append(pltpu.async_copy(src_h.at[eidx_v.at[j]], sidx_v.at[j], sem_i))
        ih.append(pltpu.async_copy(dst_h.at[eidx_v.at[j]], didx_v.at[j], sem_i))
        ih.append(pltpu.async_copy(ef_h.at[eidx_v.at[j]], efx_v.at[j], sem_i))
    pltpu.sync_copy(zrows, acc_sh.at[pl.ds(sid * zrows_pt, zrows_pt)])
    pltpu.sync_copy(dst3.at[wid], idx_v)
    plsc.subcore_barrier()

    def body(g, carry):
        pltpu.sync_copy(msgv4.at[wid, pl.ds(g * _GB, _GB)], val_v)
        handles = [
            pltpu.async_copy(val_v.at[b], acc_sh.at[idx_v.at[g * _GB + b]],
                             sem_s, add=True)
            for b in range(_GB)
        ]
        for hnd in handles:
            hnd.wait()
        return carry

    lax.fori_loop(0, n_groups, body, 0)
    for hnd in ih:
        hnd.wait()
    plsc.subcore_barrier()

    for j in range(nj):
        h1 = pltpu.async_copy(acc_sh.at[sidx_v.at[j]], sbuf_v, sem_r)
        h2 = pltpu.async_copy(acc_sh.at[didx_v.at[j]], dbuf_v, sem_r)
        h1.wait()
        h2.wait()
        pltpu.sync_copy(sbuf_v, out_s.at[sid, j, pl.ds(0, 128),
                                         pl.ds(cid * 32, 32)])
        pltpu.sync_copy(dbuf_v, out_d.at[sid, j, pl.ds(0, 128),
                                         pl.ds(cid * 32, 32)])

    @pl.when(cid == 0)
    def _():
        pltpu.sync_copy(efx_v, out_e.at[sid])


def _cls_body(sps_ref, spd_ref, ef_ref, A_ref, B_ref, C_ref, bias_ref,
              bc1_ref, Wc2_ref, bc2_ref, out_ref):
    def fin(sp):
        a = sp[:, :32] + sp[:, 32:]
        return jnp.maximum(
            a[:, :16] / jnp.maximum(a[:, 16:17], 1.0) + bias_ref[...], 0.0)

    sh = fin(sps_ref[...])
    dh = fin(spd_ref[...])
    z = jnp.maximum(
        jnp.dot(sh, A_ref[...], preferred_element_type=jnp.float32)
        + jnp.dot(dh, B_ref[...], preferred_element_type=jnp.float32)
        + jnp.dot(ef_ref[...], C_ref[...], preferred_element_type=jnp.float32)
        + bc1_ref[...], 0.0)
    out_ref[...] = jnp.dot(z, Wc2_ref[...],
                           preferred_element_type=jnp.float32) + bc2_ref[...]


def kernel(node_feats, edge_feats, edge_index, edge_indices, W1, b1, W2, b2,
           conv_bias, Wc1, bc1, Wc2, bc2):
    N, IN = node_feats.shape
    E, EF = edge_feats.shape
    H = conv_bias.shape[0]
    K = W1.shape[1]
    HI = W2.shape[1]
    NSUP = edge_indices.shape[0]
    OUT = Wc2.shape[1]
    HW = 32

    src = edge_index[0]
    dst = edge_index[1]

    BE = 960
    E_pad = ((E + _NW * 128 - 1) // (_NW * 128)) * (_NW * 128)
    assert E_pad % BE == 0
    rows_per_tile = E_pad // _NW
    n_chunks = rows_per_tile // 128

    R = (jnp.arange(HI)[None, :] // H == jnp.arange(IN)[:, None]).astype(jnp.float32)
    S = (jnp.arange(HI)[:, None] % H == jnp.arange(H)[None, :]).astype(jnp.float32)
    S = jnp.concatenate([S, jnp.zeros((HI, HW - H), jnp.float32)], axis=1)
    F = (jnp.arange(HW)[None, :] == H).astype(jnp.float32)

    src3 = jnp.pad(src, (0, E_pad - E)).reshape(_NW, n_chunks, 128)
    xsgather = functools.partial(
        pl.kernel,
        mesh=plsc.VectorSubcoreMesh(core_axis_name="c", subcore_axis_name="s"),
        compiler_params=pltpu.CompilerParams(use_tc_tiling_on_sc=False),
        out_type=jax.ShapeDtypeStruct((_NW, n_chunks, 128, IN), jnp.float32),
        scratch_types=[
            pltpu.VMEM((n_chunks, 128), jnp.int32),
            pltpu.VMEM((_GBX, 128, IN), jnp.float32),
            pltpu.SemaphoreType.DMA,
        ],
    )(functools.partial(_xs_gather_body, n_chunks // _GBX))
    xs_p = xsgather(node_feats, src3).reshape(E_pad, IN)

    grid = (E_pad // BE,)
    msgv = pl.pallas_call(
        functools.partial(_mlp_msg_body, E, BE),
        grid=grid,
        in_specs=[
            pl.BlockSpec((BE, EF), lambda i: (i, 0)),
            pl.BlockSpec((BE, IN), lambda i: (i, 0)),
            pl.BlockSpec((EF, K), lambda i: (0, 0)),
            pl.BlockSpec((1, K), lambda i: (0, 0)),
            pl.BlockSpec((K, HI), lambda i: (0, 0)),
            pl.BlockSpec((1, HI), lambda i: (0, 0)),
            pl.BlockSpec((IN, HI), lambda i: (0, 0)),
            pl.BlockSpec((HI, HW), lambda i: (0, 0)),
            pl.BlockSpec((1, HW), lambda i: (0, 0)),
        ],
        out_specs=pl.BlockSpec((BE, HW), lambda i: (i, 0)),
        out_shape=jax.ShapeDtypeStruct((E_pad, HW), jnp.float32),
    )(edge_feats, xs_p, W1, b1.reshape(1, K), W2.astype(jnp.bfloat16),
      b2.reshape(1, HI), R, S.astype(jnp.bfloat16), F)

    N_pad = ((N + _NS * 8 - 1) // (_NS * 8)) * (_NS * 8)
    zrows_pt = N_pad // _NS
    dst3 = jnp.pad(dst, (0, E_pad - E)).reshape(_NW, n_chunks, 128)
    msgv4 = msgv.reshape(_NW, n_chunks, 128, HW)
    zrows = jnp.zeros((zrows_pt, HW), jnp.float32)

    NS_pad = _NS * _NJ * 128
    eidx3 = jnp.pad(edge_indices, (0, NS_pad - NSUP)).reshape(_NS, _NJ, 128)
    agg = functools.partial(
        pl.kernel,
        mesh=plsc.VectorSubcoreMesh(core_axis_name="c", subcore_axis_name="s"),
        compiler_params=pltpu.CompilerParams(use_tc_tiling_on_sc=False),
        out_type=[
            jax.ShapeDtypeStruct((_NS, _NJ, 128, 2 * HW), jnp.float32),
            jax.ShapeDtypeStruct((_NS, _NJ, 128, 2 * HW), jnp.float32),
            jax.ShapeDtypeStruct((_NS, _NJ, 128, EF), jnp.float32),
        ],
        scratch_types=[
            pltpu.VMEM((n_chunks, 128), jnp.int32),
            pltpu.VMEM((_GB, 128, HW), jnp.float32),
            pltpu.VMEM_SHARED((N_pad, HW), jnp.float32),
            pltpu.VMEM((_NJ, 128), jnp.int32),
            pltpu.VMEM((_NJ, 128), jnp.int32),
            pltpu.VMEM((_NJ, 128), jnp.int32),
            pltpu.VMEM((_NJ, 128, EF), jnp.float32),
            pltpu.VMEM((128, HW), jnp.float32),
            pltpu.VMEM((128, HW), jnp.float32),
            pltpu.SemaphoreType.DMA,
            pltpu.SemaphoreType.DMA,
            pltpu.SemaphoreType.DMA,
        ],
    )(functools.partial(_agg_body, n_chunks // _GB, zrows_pt, _NJ))
    out_s, out_d, out_e = agg(msgv4, dst3, zrows, eidx3, src, dst, edge_feats)
    sps = out_s.reshape(NS_pad, 2 * HW)
    spd = out_d.reshape(NS_pad, 2 * HW)
    efx = out_e.reshape(NS_pad, EF)

    logits_p = pl.pallas_call(
        _cls_body,
        in_specs=[
            pl.BlockSpec((NS_pad, 2 * HW), lambda: (0, 0)),
            pl.BlockSpec((NS_pad, 2 * HW), lambda: (0, 0)),
            pl.BlockSpec((NS_pad, EF), lambda: (0, 0)),
            pl.BlockSpec((H, H), lambda: (0, 0)),
            pl.BlockSpec((H, H), lambda: (0, 0)),
            pl.BlockSpec((EF, H), lambda: (0, 0)),
            pl.BlockSpec((1, H), lambda: (0, 0)),
            pl.BlockSpec((1, H), lambda: (0, 0)),
            pl.BlockSpec((H, OUT), lambda: (0, 0)),
            pl.BlockSpec((1, OUT), lambda: (0, 0)),
        ],
        out_specs=pl.BlockSpec((NS_pad, OUT), lambda: (0, 0)),
        out_shape=jax.ShapeDtypeStruct((NS_pad, OUT), jnp.float32),
    )(sps, spd, efx, Wc1[:H], Wc1[H:2 * H], Wc1[2 * H:],
      conv_bias.reshape(1, H), bc1.reshape(1, H), Wc2, bc2.reshape(1, OUT))
    return logits_p[:NSUP]
